# baseline (speedup 1.0000x reference)
import functools

import jax
import jax.numpy as jnp
from jax import lax
from jax.experimental import pallas as pl
from jax.experimental.pallas import tpu as pltpu
from jax.experimental.pallas import tpu_sc as plsc

LANES = 16


def _emb_kernel(n_rows, hidden, seq_len, rows_per_w, chunk, num_cores):
    n_chunks = rows_per_w // chunk
    vecs_per_row = hidden // LANES

    mesh = plsc.VectorSubcoreMesh(core_axis_name="c", subcore_axis_name="s")

    @functools.partial(
        pl.kernel,
        mesh=mesh,
        out_type=jax.ShapeDtypeStruct((n_rows, hidden), jnp.float32),
        scratch_types=[
            pltpu.VMEM((rows_per_w,), jnp.int32),
            pltpu.VMEM((seq_len, hidden), jnp.float32),
            pltpu.VMEM((chunk, hidden), jnp.float32),
            pltpu.SemaphoreType.DMA,
        ],
    )
    def emb(ids_hbm, tab_hbm, pos_hbm, out_hbm, idx_v, pos_v, buf_v, sem):
        wid = lax.axis_index("s") * num_cores + lax.axis_index("c")
        base = wid * rows_per_w
        pltpu.sync_copy(ids_hbm.at[pl.ds(base, rows_per_w)], idx_v)
        pltpu.sync_copy(pos_hbm, pos_v)

        def chunk_body(c, carry):
            pltpu.async_copy(
                tab_hbm.at[idx_v.at[pl.ds(c * chunk, chunk)]], buf_v, sem
            ).wait()

            poff = (c * chunk) % seq_len

            def row_body(r, carry2):
                pr = poff + r
                for j in range(vecs_per_row):
                    plsc.addupdate(
                        buf_v.at[r, pl.ds(j * LANES, LANES)],
                        pos_v[pr, pl.ds(j * LANES, LANES)],
                    )
                return carry2

            lax.fori_loop(0, chunk, row_body, 0, unroll=False)

            pltpu.sync_copy(buf_v, out_hbm.at[pl.ds(base + c * chunk, chunk)])
            return carry

        lax.fori_loop(0, n_chunks, chunk_body, 0, unroll=False)

    return emb


def kernel(input_ids, tokens_embedding, position_embedding):
    batch, seq_len = input_ids.shape
    vocab, hidden = tokens_embedding.shape
    n_rows = batch * seq_len

    info = plsc.get_sparse_core_info()
    num_workers = info.num_cores * info.num_subcores
    rows_per_w = n_rows // num_workers
    chunk = seq_len

    ids_flat = input_ids.reshape(n_rows).astype(jnp.int32)
    emb = _emb_kernel(
        n_rows, hidden, seq_len, rows_per_w, chunk, info.num_cores
    )
    out = emb(ids_flat, tokens_embedding, position_embedding)
    return out.reshape(batch, seq_len, hidden)
